# baseline (speedup 1.0000x reference)
import functools

import jax
import jax.numpy as jnp
from jax import lax
from jax.experimental import pallas as pl
from jax.experimental.pallas import tpu as pltpu
from jax.experimental.pallas import tpu_sc as plsc

N_NODES = 10000
N_EDGES = 320000
D = 128

NC = 2
NS = 16
NW = NC * NS
EPW = N_EDGES // NW
B = 80
NB = EPW // B
CH = 25
NCH = NB // CH
NBUF = 3

_mesh = plsc.VectorSubcoreMesh(core_axis_name="c", subcore_axis_name="s")


@functools.partial(
    pl.kernel,
    mesh=_mesh,
    out_type=jax.ShapeDtypeStruct((NC, N_NODES, D), jnp.float32),
    scratch_types=[
        pltpu.VMEM((CH, B), jnp.int32),
        pltpu.VMEM((CH, B), jnp.int32),
        pltpu.VMEM((B, D), jnp.float32),
        pltpu.VMEM((B, D), jnp.float32),
        pltpu.VMEM((B, D), jnp.float32),
        pltpu.VMEM_SHARED((N_NODES, D), jnp.float32),
        pltpu.SemaphoreType.DMA,
        pltpu.SemaphoreType.DMA,
        pltpu.SemaphoreType.DMA,
        pltpu.SemaphoreType.DMA,
        pltpu.SemaphoreType.DMA,
        pltpu.SemaphoreType.DMA,
    ],
)
def _message_pass(feat_hbm, idx_hbm, out_hbm,
                  src_v, dst_v, rows0, rows1, rows2, h_sh,
                  sem0, sem1, sem2, ssem0, ssem1, ssem2):
    c = lax.axis_index("c")
    s = lax.axis_index("s")
    wid = s * NC + c
    rpt = 624
    tail_base = NS * rpt
    tail = N_NODES - tail_base

    zv = jnp.zeros((16,), jnp.float32)

    def zb(j, c2):
        r = j // (D // 16)
        col = (j % (D // 16)) * 16
        rows0[r, pl.ds(col, 16)] = zv
        return c2

    lax.fori_loop(0, B * D // 16, zb, 0)
    for k in range(rpt // B):
        pltpu.sync_copy(rows0, h_sh.at[pl.ds(s * rpt + k * B, B)])
    rem = rpt - (rpt // B) * B
    pltpu.sync_copy(rows0.at[pl.ds(0, rem)],
                    h_sh.at[pl.ds(s * rpt + (rpt // B) * B, rem)])

    @pl.when(s == NS - 1)
    def _():
        pltpu.sync_copy(rows0.at[pl.ds(0, tail)],
                        h_sh.at[pl.ds(tail_base, tail)])
    plsc.subcore_barrier()

    bufs = (rows0, rows1, rows2)
    sems = (sem0, sem1, sem2)
    ssems = (ssem0, ssem1, ssem2)

    def start(i, b):
        pltpu.async_copy(feat_hbm.at[src_v.at[i]], bufs[b], sems[b])

    def wait(b):
        pltpu.make_async_copy(feat_hbm.at[src_v.at[0]], bufs[b], sems[b]).wait()

    def scatter_start(i, b):
        pltpu.async_copy(bufs[b], h_sh.at[dst_v.at[i]], ssems[b], add=True)

    def scatter_wait(b):
        pltpu.make_async_copy(bufs[b], h_sh.at[dst_v.at[0]], ssems[b]).wait()

    def chunk(ch, carry):
        pltpu.sync_copy(idx_hbm.at[0].at[wid].at[ch], src_v)
        pltpu.sync_copy(idx_hbm.at[1].at[wid].at[ch], dst_v)
        for i in range(NBUF - 1):
            start(i, i % NBUF)
        for i in range(CH):
            j = i + NBUF - 1
            if j < CH:
                b = j % NBUF
                if j >= NBUF:
                    scatter_wait(b)
                start(j, b)
            wait(i % NBUF)
            scatter_start(i, i % NBUF)
        for i in range(CH - NBUF, CH):
            scatter_wait(i % NBUF)
        return carry

    lax.fori_loop(0, NCH, chunk, 0)

    plsc.subcore_barrier()
    pltpu.sync_copy(h_sh.at[pl.ds(s * rpt, rpt)],
                    out_hbm.at[c].at[pl.ds(s * rpt, rpt)])

    @pl.when(s == NS - 1)
    def _():
        pltpu.sync_copy(h_sh.at[pl.ds(tail_base, tail)],
                        out_hbm.at[c].at[pl.ds(tail_base, tail)])


def _linear_body(p_ref, w_ref, b_ref, o_ref):
    h = p_ref[0] + p_ref[1]
    o_ref[...] = lax.dot_general(
        h, w_ref[...], (((1,), (1,)), ((), ())),
        preferred_element_type=jnp.float32) + b_ref[...]


def _linear(part, W, b):
    return pl.pallas_call(
        _linear_body,
        in_specs=[
            pl.BlockSpec((NC, N_NODES, D), lambda: (0, 0, 0)),
            pl.BlockSpec((D, D), lambda: (0, 0)),
            pl.BlockSpec((1, D), lambda: (0, 0)),
        ],
        out_specs=pl.BlockSpec((N_NODES, D), lambda: (0, 0)),
        out_shape=jax.ShapeDtypeStruct((N_NODES, D), jnp.float32),
    )(part, W, b.reshape(1, D))


@jax.jit
def kernel(feature, edge_index, W, b):
    idx = edge_index.astype(jnp.int32).reshape(2, NW, NCH, CH, B)
    part = _message_pass(feature, idx)
    return _linear(part, W, b)


# speedup vs baseline: 1.0811x; 1.0811x over previous
import functools

import jax
import jax.numpy as jnp
from jax import lax
from jax.experimental import pallas as pl
from jax.experimental.pallas import tpu as pltpu
from jax.experimental.pallas import tpu_sc as plsc

N_NODES = 10000
N_EDGES = 320000
D = 128

NC = 2
NS = 16
NW = NC * NS
EPW = N_EDGES // NW
B = 80
NB = EPW // B
CH = 5
NCH = NB // CH
NBUF = 3
SEG = CH * NBUF

_mesh = plsc.VectorSubcoreMesh(core_axis_name="c", subcore_axis_name="s")


@functools.partial(
    pl.kernel,
    mesh=_mesh,
    out_type=jax.ShapeDtypeStruct((NC, N_NODES, D), jnp.float32),
    scratch_types=[
        pltpu.VMEM((CH, B), jnp.int32),
        pltpu.VMEM((CH, B), jnp.int32),
        pltpu.VMEM((CH, B), jnp.int32),
        pltpu.VMEM((CH, B), jnp.int32),
        pltpu.VMEM((CH, B), jnp.int32),
        pltpu.VMEM((CH, B), jnp.int32),
        pltpu.VMEM((B, D), jnp.float32),
        pltpu.VMEM((B, D), jnp.float32),
        pltpu.VMEM((B, D), jnp.float32),
        pltpu.VMEM_SHARED((N_NODES, D), jnp.float32),
        pltpu.SemaphoreType.DMA,
        pltpu.SemaphoreType.DMA,
        pltpu.SemaphoreType.DMA,
        pltpu.SemaphoreType.DMA,
        pltpu.SemaphoreType.DMA,
        pltpu.SemaphoreType.DMA,
        pltpu.SemaphoreType.DMA,
        pltpu.SemaphoreType.DMA,
        pltpu.SemaphoreType.DMA,
    ],
)
def _message_pass(feat_hbm, idx_hbm, out_hbm,
                  src0, dst0, src1, dst1, src2, dst2,
                  rows0, rows1, rows2, h_sh,
                  sem0, sem1, sem2, ssem0, ssem1, ssem2,
                  isem0, isem1, isem2):
    c = lax.axis_index("c")
    s = lax.axis_index("s")
    wid = s * NC + c
    rpt = 624
    tail_base = NS * rpt
    tail = N_NODES - tail_base

    zv = jnp.zeros((16,), jnp.float32)

    def zb(j, c2):
        r = j // (D // 16)
        col = (j % (D // 16)) * 16
        rows0[r, pl.ds(col, 16)] = zv
        return c2

    lax.fori_loop(0, B * D // 16, zb, 0)
    for k in range(rpt // B):
        pltpu.sync_copy(rows0, h_sh.at[pl.ds(s * rpt + k * B, B)])
    rem = rpt - (rpt // B) * B
    pltpu.sync_copy(rows0.at[pl.ds(0, rem)],
                    h_sh.at[pl.ds(s * rpt + (rpt // B) * B, rem)])

    @pl.when(s == NS - 1)
    def _():
        pltpu.sync_copy(rows0.at[pl.ds(0, tail)],
                        h_sh.at[pl.ds(tail_base, tail)])
    plsc.subcore_barrier()

    bufs = (rows0, rows1, rows2)
    sems = (sem0, sem1, sem2)
    ssems = (ssem0, ssem1, ssem2)
    srcs = (src0, src1, src2)
    dsts = (dst0, dst1, dst2)
    isems = (isem0, isem1, isem2)

    def iload(ch, st):
        pltpu.async_copy(idx_hbm.at[0].at[wid].at[ch], srcs[st], isems[st])
        pltpu.async_copy(idx_hbm.at[1].at[wid].at[ch], dsts[st], isems[st])

    def iwait(st):
        pltpu.make_async_copy(idx_hbm.at[0].at[wid].at[0], srcs[st],
                              isems[st]).wait()
        pltpu.make_async_copy(idx_hbm.at[1].at[wid].at[0], dsts[st],
                              isems[st]).wait()

    def start(st, r, b):
        pltpu.async_copy(feat_hbm.at[srcs[st].at[r]], bufs[b], sems[b])

    def wait(b):
        pltpu.make_async_copy(feat_hbm.at[src0.at[0]], bufs[b], sems[b]).wait()

    def scatter_start(st, r, b):
        pltpu.async_copy(bufs[b], h_sh.at[dsts[st].at[r]], ssems[b], add=True)

    def scatter_wait(b):
        pltpu.make_async_copy(bufs[b], h_sh.at[dst0.at[0]], ssems[b]).wait()

    def emit_steps(bc, length, tail_chunks=None, first=False):
        for o in range(length):
            j = o + NBUF - 1
            j_valid = tail_chunks is None or j <= length - 1
            if j_valid and not (first and o == 0):
                scatter_wait(j % NBUF)
            if o % CH == 0:
                co = o // CH
                skip = (first and co == 0) or (
                    tail_chunks is not None and co + 2 >= tail_chunks)
                if not skip:
                    iload(bc + co + 2, (co + 2) % 3)
            if j_valid:
                if j % CH == 0:
                    iwait((j // CH) % 3)
                start((j // CH) % 3, j % CH, j % NBUF)
            wait(o % NBUF)
            scatter_start((o // CH) % 3, o % CH, o % NBUF)

    iload(0, 0)
    iload(1, 1)
    iload(2, 2)
    iwait(0)
    start(0, 0, 0)
    start(0, 1, 1)

    emit_steps(0, SEG, first=True)

    def body(it, carry):
        emit_steps(3 * it, SEG)
        return carry

    lax.fori_loop(1, 7, body, 0)

    emit_steps(21, (NCH - 21) * CH, tail_chunks=NCH - 21)

    for g in range(NB - NBUF, NB):
        scatter_wait(g % NBUF)

    plsc.subcore_barrier()
    pltpu.sync_copy(h_sh.at[pl.ds(s * rpt, rpt)],
                    out_hbm.at[c].at[pl.ds(s * rpt, rpt)])

    @pl.when(s == NS - 1)
    def _():
        pltpu.sync_copy(h_sh.at[pl.ds(tail_base, tail)],
                        out_hbm.at[c].at[pl.ds(tail_base, tail)])


def _linear_body(p_ref, w_ref, b_ref, o_ref):
    h = p_ref[0] + p_ref[1]
    o_ref[...] = lax.dot_general(
        h, w_ref[...], (((1,), (1,)), ((), ())),
        preferred_element_type=jnp.float32) + b_ref[...]


def _linear(part, W, b):
    return pl.pallas_call(
        _linear_body,
        in_specs=[
            pl.BlockSpec((NC, N_NODES, D), lambda: (0, 0, 0)),
            pl.BlockSpec((D, D), lambda: (0, 0)),
            pl.BlockSpec((1, D), lambda: (0, 0)),
        ],
        out_specs=pl.BlockSpec((N_NODES, D), lambda: (0, 0)),
        out_shape=jax.ShapeDtypeStruct((N_NODES, D), jnp.float32),
    )(part, W, b.reshape(1, D))


@jax.jit
def kernel(feature, edge_index, W, b):
    idx = edge_index.astype(jnp.int32).reshape(2, NW, NCH, CH, B)
    part = _message_pass(feature, idx)
    return _linear(part, W, b)


# speedup vs baseline: 1.0904x; 1.0086x over previous
import functools

import jax
import jax.numpy as jnp
from jax import lax
from jax.experimental import pallas as pl
from jax.experimental.pallas import tpu as pltpu
from jax.experimental.pallas import tpu_sc as plsc

N_NODES = 10000
N_EDGES = 320000
D = 128

NC = 2
NS = 16
NW = NC * NS
EPW = N_EDGES // NW
B = 80
NB = EPW // B
CH = 5
NCH = NB // CH
NBUF = 3
SEG = CH * NBUF

_mesh = plsc.VectorSubcoreMesh(core_axis_name="c", subcore_axis_name="s")


@functools.partial(
    pl.kernel,
    mesh=_mesh,
    out_type=jax.ShapeDtypeStruct((NC, N_NODES, D), jnp.float32),
    scratch_types=[
        pltpu.VMEM((CH, B), jnp.int32),
        pltpu.VMEM((CH, B), jnp.int32),
        pltpu.VMEM((CH, B), jnp.int32),
        pltpu.VMEM((CH, B), jnp.int32),
        pltpu.VMEM((CH, B), jnp.int32),
        pltpu.VMEM((CH, B), jnp.int32),
        pltpu.VMEM((B, D), jnp.float32),
        pltpu.VMEM((B, D), jnp.float32),
        pltpu.VMEM((B, D), jnp.float32),
        pltpu.VMEM_SHARED((N_NODES, D), jnp.float32),
        pltpu.SemaphoreType.DMA,
        pltpu.SemaphoreType.DMA,
        pltpu.SemaphoreType.DMA,
        pltpu.SemaphoreType.DMA,
        pltpu.SemaphoreType.DMA,
        pltpu.SemaphoreType.DMA,
        pltpu.SemaphoreType.DMA,
        pltpu.SemaphoreType.DMA,
        pltpu.SemaphoreType.DMA,
    ],
)
def _message_pass(feat_hbm, idx_hbm, out_hbm,
                  src0, dst0, src1, dst1, src2, dst2,
                  rows0, rows1, rows2, h_sh,
                  sem0, sem1, sem2, ssem0, ssem1, ssem2,
                  isem0, isem1, isem2):
    c = lax.axis_index("c")
    s = lax.axis_index("s")
    wid = s * NC + c
    rpt = 624
    tail_base = NS * rpt
    tail = N_NODES - tail_base

    zv = jnp.zeros((16,), jnp.float32)

    def zb(j, c2):
        r = j // (D // 16)
        col = (j % (D // 16)) * 16
        rows0[r, pl.ds(col, 16)] = zv
        return c2

    lax.fori_loop(0, B * D // 16, zb, 0)

    bufs = (rows0, rows1, rows2)
    sems = (sem0, sem1, sem2)
    ssems = (ssem0, ssem1, ssem2)
    srcs = (src0, src1, src2)
    dsts = (dst0, dst1, dst2)
    isems = (isem0, isem1, isem2)

    def iload(ch, st):
        pltpu.async_copy(idx_hbm.at[0].at[wid].at[ch], srcs[st], isems[st])
        pltpu.async_copy(idx_hbm.at[1].at[wid].at[ch], dsts[st], isems[st])

    def iwait(st):
        pltpu.make_async_copy(idx_hbm.at[0].at[wid].at[0], srcs[st],
                              isems[st]).wait()
        pltpu.make_async_copy(idx_hbm.at[1].at[wid].at[0], dsts[st],
                              isems[st]).wait()

    def start(st, r, b):
        pltpu.async_copy(feat_hbm.at[srcs[st].at[r]], bufs[b], sems[b])

    def wait(b):
        pltpu.make_async_copy(feat_hbm.at[src0.at[0]], bufs[b], sems[b]).wait()

    def scatter_start(st, r, b):
        pltpu.async_copy(bufs[b], h_sh.at[dsts[st].at[r]], ssems[b], add=True)

    def scatter_wait(b):
        pltpu.make_async_copy(bufs[b], h_sh.at[dst0.at[0]], ssems[b]).wait()

    def emit_steps(bc, length, tail_chunks=None, first=False):
        for o in range(length):
            j = o + NBUF - 1
            j_valid = tail_chunks is None or j <= length - 1
            if j_valid and not (first and o == 0):
                scatter_wait(j % NBUF)
            if o % CH == 0:
                co = o // CH
                skip = (first and co == 0) or (
                    tail_chunks is not None and co + 2 >= tail_chunks)
                if not skip:
                    iload(bc + co + 2, (co + 2) % 3)
            if j_valid:
                if j % CH == 0:
                    iwait((j // CH) % 3)
                start((j // CH) % 3, j % CH, j % NBUF)
            wait(o % NBUF)
            scatter_start((o // CH) % 3, o % CH, o % NBUF)

    iload(0, 0)
    iload(1, 1)
    iload(2, 2)
    nfull = rpt // B
    rem = rpt - nfull * B
    for k in range(nfull):
        pltpu.async_copy(rows0, h_sh.at[pl.ds(s * rpt + k * B, B)],
                         ssems[k % 3])
    pltpu.async_copy(rows0.at[pl.ds(0, rem)],
                     h_sh.at[pl.ds(s * rpt + nfull * B, rem)], ssems[1])

    @pl.when(s == NS - 1)
    def _():
        pltpu.async_copy(rows0.at[pl.ds(0, tail)],
                         h_sh.at[pl.ds(tail_base, tail)], ssems[2])

    for k in range(nfull):
        pltpu.make_async_copy(rows0, h_sh.at[pl.ds(s * rpt + k * B, B)],
                              ssems[k % 3]).wait()
    pltpu.make_async_copy(rows0.at[pl.ds(0, rem)],
                          h_sh.at[pl.ds(s * rpt + nfull * B, rem)],
                          ssems[1]).wait()

    @pl.when(s == NS - 1)
    def _():
        pltpu.make_async_copy(rows0.at[pl.ds(0, tail)],
                              h_sh.at[pl.ds(tail_base, tail)],
                              ssems[2]).wait()

    plsc.subcore_barrier()
    iwait(0)
    start(0, 0, 0)
    start(0, 1, 1)

    emit_steps(0, SEG, first=True)

    def body(it, carry):
        emit_steps(3 * it, SEG)
        return carry

    lax.fori_loop(1, 7, body, 0)

    emit_steps(21, (NCH - 21) * CH, tail_chunks=NCH - 21)

    for g in range(NB - NBUF, NB):
        scatter_wait(g % NBUF)

    plsc.subcore_barrier()
    pltpu.sync_copy(h_sh.at[pl.ds(s * rpt, rpt)],
                    out_hbm.at[c].at[pl.ds(s * rpt, rpt)])

    @pl.when(s == NS - 1)
    def _():
        pltpu.sync_copy(h_sh.at[pl.ds(tail_base, tail)],
                        out_hbm.at[c].at[pl.ds(tail_base, tail)])


def _linear_body(p_ref, w_ref, b_ref, o_ref):
    h = p_ref[0] + p_ref[1]
    o_ref[...] = lax.dot_general(
        h, w_ref[...], (((1,), (1,)), ((), ())),
        preferred_element_type=jnp.float32) + b_ref[...]


def _linear(part, W, b):
    return pl.pallas_call(
        _linear_body,
        in_specs=[
            pl.BlockSpec((NC, N_NODES, D), lambda: (0, 0, 0)),
            pl.BlockSpec((D, D), lambda: (0, 0)),
            pl.BlockSpec((1, D), lambda: (0, 0)),
        ],
        out_specs=pl.BlockSpec((N_NODES, D), lambda: (0, 0)),
        out_shape=jax.ShapeDtypeStruct((N_NODES, D), jnp.float32),
    )(part, W, b.reshape(1, D))


@jax.jit
def kernel(feature, edge_index, W, b):
    idx = edge_index.astype(jnp.int32).reshape(2, NW, NCH, CH, B)
    part = _message_pass(feature, idx)
    return _linear(part, W, b)
